# baseline (speedup 1.0000x reference)
import jax
import jax.numpy as jnp
from jax import lax
from jax.experimental import pallas as pl
from jax.experimental.pallas import tpu as pltpu
from jax.experimental.pallas import tpu_sc as plsc

_B, _N, _C, _K = 4, 2048, 32, 16
_CD = _C * 3
_RB = 256

_NC, _NS = 2, 16
_NW = _NC * _NS
_WPB = _NW // _B
_RPW = _CD // _WPB


def _knn_body(xyz_ref, xyzT_ref, idxT_ref, dist_ref):
    xall = xyz_ref[0]
    xTr = xyzT_ref[0]
    innerT = lax.dot_general(
        xall.astype(jnp.bfloat16), xTr.astype(jnp.bfloat16),
        (((1,), (0,)), ((), ())),
        preferred_element_type=jnp.float32)
    sq_all = (xall[:, 0:1] ** 2 + xall[:, 1:2] ** 2
              + xall[:, 2:3] ** 2)
    sq_r = xTr[0:1, :] ** 2 + xTr[1:2, :] ** 2 + xTr[2:3, :] ** 2
    dist_ref[...] = sq_all - 2.0 * innerT + sq_r
    iota8 = lax.broadcasted_iota(jnp.int32, (8, _RB), 0)
    inf8 = jnp.full((8, _RB), jnp.inf, jnp.float32)
    big8 = jnp.full((8, _RB), _N, jnp.int32)
    am = jnp.full((1, _RB), -1, jnp.int32)
    rows = []
    for _ in range(_K):
        def body(r, carry, am=am):
            accv, acci = carry
            tile = dist_ref[pl.ds(r * 8, 8), :]
            ridx = iota8 + r * 8
            masked = jnp.where(ridx == am, jnp.inf, tile)
            dist_ref[pl.ds(r * 8, 8), :] = masked
            c = masked < accv
            accv = jnp.where(c, masked, accv)
            acci = jnp.where(c, ridx, acci)
            return accv, acci
        accv, acci = lax.fori_loop(0, _N // 8, body, (inf8, big8),
                                   unroll=2)
        m = jnp.min(accv, axis=0, keepdims=True)
        am = jnp.min(jnp.where(accv == m, acci, _N),
                     axis=0, keepdims=True)
        rows.append(am)
    idxT_ref[0] = jnp.concatenate(rows, axis=0)


def _knn_idx(points_xyz, xyzT):
    grid = (_B, _N // _RB)
    return pl.pallas_call(
        _knn_body,
        grid=grid,
        in_specs=[
            pl.BlockSpec((1, _N, 3), lambda b, r: (b, 0, 0)),
            pl.BlockSpec((1, 3, _RB), lambda b, r: (b, 0, r)),
        ],
        out_specs=pl.BlockSpec((1, _K, _RB), lambda b, r: (b, 0, r)),
        out_shape=jax.ShapeDtypeStruct((_B, _K, _N), jnp.int32),
        scratch_shapes=[pltpu.VMEM((_N, _RB), jnp.float32)],
    )(points_xyz, xyzT)


def _sc_ctr_body(fts_hbm, gf_hbm, in_v, sem_in, sem_c0, sem_c1, sem_c2):
    w = lax.axis_index("s") * _NC + lax.axis_index("c")
    b = w // _WPB
    slot = w % _WPB
    base = slot * _RPW
    sem_c = (sem_c0, sem_c1, sem_c2)
    hin, hctr = {}, {}
    hin[0] = pltpu.async_copy(fts_hbm.at[b, pl.ds(base, 1)], in_v.at[0],
                              sem_in)
    for t in range(_RPW):
        bi3 = t % 3
        hin[t].wait()
        if t + 1 < _RPW:
            if t - 2 in hctr:
                for x in hctr.pop(t - 2):
                    x.wait()
            hin[t + 1] = pltpu.async_copy(
                fts_hbm.at[b, pl.ds(base + t + 1, 1)],
                in_v.at[(t + 1) % 3], sem_in)
        hctr[t] = [pltpu.async_copy(in_v.at[bi3],
                                    gf_hbm.at[b, 1, base + t, pl.ds(k, 1)],
                                    sem_c[bi3])
                   for k in range(_K)]
    for t in sorted(hctr):
        for x in hctr[t]:
            x.wait()


def _sc_ctr(fts, gf_ref):
    mesh = plsc.VectorSubcoreMesh(core_axis_name="c", subcore_axis_name="s",
                                  num_cores=_NC, num_subcores=_NS)
    f = pl.kernel(
        _sc_ctr_body,
        out_type=(),
        mesh=mesh,
        compiler_params=pltpu.CompilerParams(needs_layout_passes=False),
        scratch_types=[
            pltpu.VMEM((3, 1, _N), jnp.float32),
            pltpu.SemaphoreType.DMA,
            pltpu.SemaphoreType.DMA,
            pltpu.SemaphoreType.DMA,
            pltpu.SemaphoreType.DMA,
        ],
    )
    return f(fts, gf_ref)


def _sc_body(fts_hbm, xyzT_hbm, idxT_hbm, gf_hbm, gx_hbm,
             idx_v, in_v, diff_v, sem_in, sem_d0, sem_d1):
    w = lax.axis_index("s") * _NC + lax.axis_index("c")
    b = w // _WPB
    slot = w % _WPB
    base = slot * _RPW
    sem_d = (sem_d0, sem_d1)

    pltpu.sync_copy(idxT_hbm.at[b], idx_v)

    def run_gather(bi3, bi2):
        def one(i, _):
            n0 = i * 16
            cvec = in_v[bi3, 0, pl.ds(n0, 16)]
            for k in range(_K):
                iv = idx_v[k, pl.ds(n0, 16)]
                g = plsc.load_gather(in_v.at[bi3, 0], [iv])
                diff_v[bi2, k, pl.ds(n0, 16)] = g - cvec
            return 0
        lax.fori_loop(0, _N // 16, one, 0)

    hin, hdiff = {}, {}
    hin[0] = pltpu.async_copy(fts_hbm.at[b, pl.ds(base, 1)], in_v.at[0], sem_in)
    for t in range(_RPW):
        bi3, bi2 = t % 3, t % 2
        hin[t].wait()
        if t + 1 < _RPW:
            hin[t + 1] = pltpu.async_copy(
                fts_hbm.at[b, pl.ds(base + t + 1, 1)], in_v.at[(t + 1) % 3], sem_in)
        if t - 2 in hdiff:
            hdiff.pop(t - 2).wait()
        run_gather(bi3, bi2)
        hdiff[t] = pltpu.async_copy(
            diff_v.at[bi2], gf_hbm.at[b, 0, base + t], sem_d[bi2])
    for t in sorted(hdiff):
        hdiff[t].wait()

    @pl.when(slot < 3)
    def _():
        pltpu.sync_copy(xyzT_hbm.at[b, pl.ds(slot, 1)], in_v.at[0])
        run_gather(0, 0)
        pltpu.sync_copy(diff_v.at[0], gx_hbm.at[b, slot])


def _sc_gather(fts, xyzT, idxT, gf_ref):
    mesh = plsc.VectorSubcoreMesh(core_axis_name="c", subcore_axis_name="s",
                                  num_cores=_NC, num_subcores=_NS)
    f = pl.kernel(
        _sc_body,
        out_type=[
            jax.ShapeDtypeStruct((_B, 3, _K, _N), jnp.float32),
        ],
        mesh=mesh,
        compiler_params=pltpu.CompilerParams(needs_layout_passes=False),
        scratch_types=[
            pltpu.VMEM((_K, _N), jnp.int32),
            pltpu.VMEM((3, 1, _N), jnp.float32),
            pltpu.VMEM((2, _K, _N), jnp.float32),
            pltpu.SemaphoreType.DMA,
            pltpu.SemaphoreType.DMA,
            pltpu.SemaphoreType.DMA,
        ],
    )
    return f(fts, xyzT, idxT, gf_ref)


def kernel(points_xyz, points_fts):
    xyzT = jnp.transpose(points_xyz, (0, 2, 1))
    fts = points_fts.reshape(_B, _CD, _N)
    gf_ref = jax.empty_ref(
        jax.ShapeDtypeStruct((_B, 2, _CD, _K, _N), jnp.float32))
    _sc_ctr(fts, gf_ref)
    idxT = _knn_idx(points_xyz, xyzT)
    [gx] = _sc_gather(fts, xyzT, idxT, gf_ref)
    gf = jax.freeze(gf_ref)
    group_fts = jnp.transpose(gf.reshape(_B, 2 * _C, 3, _K, _N),
                              (0, 1, 2, 4, 3))
    group_xyz = jnp.transpose(gx, (0, 3, 2, 1))
    new_fts = jnp.concatenate([points_fts, jnp.zeros_like(points_fts)],
                              axis=1)
    return (group_xyz, group_fts, points_xyz, new_fts)


# speedup vs baseline: 1.9890x; 1.9890x over previous
import jax
import jax.numpy as jnp
from jax import lax
from jax.experimental import pallas as pl
from jax.experimental.pallas import tpu as pltpu
from jax.experimental.pallas import tpu_sc as plsc

_B, _N, _C, _K = 4, 2048, 32, 16
_CD = _C * 3
_RB = 256

_NC, _NS = 2, 16
_NW = _NC * _NS
_WPB = _NW // _B
_RPW = _CD // _WPB


def _knn_body(xyz_ref, xyzT_ref, idxT_ref, dist_ref):
    xall = xyz_ref[0]
    xTr = xyzT_ref[0]
    innerT = lax.dot_general(
        xall.astype(jnp.bfloat16), xTr.astype(jnp.bfloat16),
        (((1,), (0,)), ((), ())),
        preferred_element_type=jnp.float32)
    sq_all = (xall[:, 0:1] ** 2 + xall[:, 1:2] ** 2
              + xall[:, 2:3] ** 2)
    sq_r = xTr[0:1, :] ** 2 + xTr[1:2, :] ** 2 + xTr[2:3, :] ** 2
    dist = sq_all - 2.0 * innerT + sq_r
    del dist_ref
    iota = lax.broadcasted_iota(jnp.int32, (_N, _RB), 0)
    am = jnp.full((1, _RB), -1, jnp.int32)
    rows = []
    for _ in range(_K):
        dist = jnp.where(iota == am, jnp.inf, dist)
        am = jnp.argmin(dist, axis=0).astype(jnp.int32)[None, :]
        rows.append(am)
    idxT_ref[0] = jnp.concatenate(rows, axis=0)


def _knn_idx(points_xyz, xyzT):
    grid = (_B, _N // _RB)
    return pl.pallas_call(
        _knn_body,
        grid=grid,
        in_specs=[
            pl.BlockSpec((1, _N, 3), lambda b, r: (b, 0, 0)),
            pl.BlockSpec((1, 3, _RB), lambda b, r: (b, 0, r)),
        ],
        out_specs=pl.BlockSpec((1, _K, _RB), lambda b, r: (b, 0, r)),
        out_shape=jax.ShapeDtypeStruct((_B, _K, _N), jnp.int32),
        scratch_shapes=[pltpu.VMEM((_N, _RB), jnp.float32)],
    )(points_xyz, xyzT)


def _sc_ctr_body(fts_hbm, gf_hbm, in_v, sem_in, sem_c0, sem_c1, sem_c2):
    w = lax.axis_index("s") * _NC + lax.axis_index("c")
    b = w // _WPB
    slot = w % _WPB
    base = slot * _RPW
    sem_c = (sem_c0, sem_c1, sem_c2)
    hin, hctr = {}, {}
    hin[0] = pltpu.async_copy(fts_hbm.at[b, pl.ds(base, 1)], in_v.at[0],
                              sem_in)
    for t in range(_RPW):
        bi3 = t % 3
        hin[t].wait()
        if t + 1 < _RPW:
            if t - 2 in hctr:
                for x in hctr.pop(t - 2):
                    x.wait()
            hin[t + 1] = pltpu.async_copy(
                fts_hbm.at[b, pl.ds(base + t + 1, 1)],
                in_v.at[(t + 1) % 3], sem_in)
        hctr[t] = [pltpu.async_copy(in_v.at[bi3],
                                    gf_hbm.at[b, 1, base + t, pl.ds(k, 1)],
                                    sem_c[bi3])
                   for k in range(_K)]
    for t in sorted(hctr):
        for x in hctr[t]:
            x.wait()


def _sc_ctr(fts, gf_ref):
    mesh = plsc.VectorSubcoreMesh(core_axis_name="c", subcore_axis_name="s",
                                  num_cores=_NC, num_subcores=_NS)
    f = pl.kernel(
        _sc_ctr_body,
        out_type=(),
        mesh=mesh,
        compiler_params=pltpu.CompilerParams(needs_layout_passes=False),
        scratch_types=[
            pltpu.VMEM((3, 1, _N), jnp.float32),
            pltpu.SemaphoreType.DMA,
            pltpu.SemaphoreType.DMA,
            pltpu.SemaphoreType.DMA,
            pltpu.SemaphoreType.DMA,
        ],
    )
    return f(fts, gf_ref)


def _sc_body(fts_hbm, xyzT_hbm, idxT_hbm, gf_hbm, gx_hbm,
             idx_v, in_v, diff_v, sem_in, sem_d0, sem_d1):
    w = lax.axis_index("s") * _NC + lax.axis_index("c")
    b = w // _WPB
    slot = w % _WPB
    base = slot * _RPW
    sem_d = (sem_d0, sem_d1)

    pltpu.sync_copy(idxT_hbm.at[b], idx_v)

    def run_gather(bi3, bi2):
        def one(i, _):
            n0 = i * 16
            cvec = in_v[bi3, 0, pl.ds(n0, 16)]
            for k in range(_K):
                iv = idx_v[k, pl.ds(n0, 16)]
                g = plsc.load_gather(in_v.at[bi3, 0], [iv])
                diff_v[bi2, k, pl.ds(n0, 16)] = g - cvec
            return 0
        lax.fori_loop(0, _N // 16, one, 0)

    hin, hdiff = {}, {}
    hin[0] = pltpu.async_copy(fts_hbm.at[b, pl.ds(base, 1)], in_v.at[0], sem_in)
    for t in range(_RPW):
        bi3, bi2 = t % 3, t % 2
        hin[t].wait()
        if t + 1 < _RPW:
            hin[t + 1] = pltpu.async_copy(
                fts_hbm.at[b, pl.ds(base + t + 1, 1)], in_v.at[(t + 1) % 3], sem_in)
        if t - 2 in hdiff:
            hdiff.pop(t - 2).wait()
        run_gather(bi3, bi2)
        hdiff[t] = pltpu.async_copy(
            diff_v.at[bi2], gf_hbm.at[b, 0, base + t], sem_d[bi2])
    for t in sorted(hdiff):
        hdiff[t].wait()

    @pl.when(slot < 3)
    def _():
        pltpu.sync_copy(xyzT_hbm.at[b, pl.ds(slot, 1)], in_v.at[0])
        run_gather(0, 0)
        pltpu.sync_copy(diff_v.at[0], gx_hbm.at[b, slot])


def _sc_gather(fts, xyzT, idxT, gf_ref):
    mesh = plsc.VectorSubcoreMesh(core_axis_name="c", subcore_axis_name="s",
                                  num_cores=_NC, num_subcores=_NS)
    f = pl.kernel(
        _sc_body,
        out_type=[
            jax.ShapeDtypeStruct((_B, 3, _K, _N), jnp.float32),
        ],
        mesh=mesh,
        compiler_params=pltpu.CompilerParams(needs_layout_passes=False),
        scratch_types=[
            pltpu.VMEM((_K, _N), jnp.int32),
            pltpu.VMEM((3, 1, _N), jnp.float32),
            pltpu.VMEM((2, _K, _N), jnp.float32),
            pltpu.SemaphoreType.DMA,
            pltpu.SemaphoreType.DMA,
            pltpu.SemaphoreType.DMA,
        ],
    )
    return f(fts, xyzT, idxT, gf_ref)


def kernel(points_xyz, points_fts):
    xyzT = jnp.transpose(points_xyz, (0, 2, 1))
    fts = points_fts.reshape(_B, _CD, _N)
    gf_ref = jax.empty_ref(
        jax.ShapeDtypeStruct((_B, 2, _CD, _K, _N), jnp.float32))
    _sc_ctr(fts, gf_ref)
    idxT = _knn_idx(points_xyz, xyzT)
    [gx] = _sc_gather(fts, xyzT, idxT, gf_ref)
    gf = jax.freeze(gf_ref)
    group_fts = jnp.transpose(gf.reshape(_B, 2 * _C, 3, _K, _N),
                              (0, 1, 2, 4, 3))
    group_xyz = jnp.transpose(gx, (0, 3, 2, 1))
    new_fts = jnp.concatenate([points_fts, jnp.zeros_like(points_fts)],
                              axis=1)
    return (group_xyz, group_fts, points_xyz, new_fts)
